# Initial kernel scaffold; baseline (speedup 1.0000x reference)
import jax
import jax.numpy as jnp
from jax.experimental import pallas as pl


def kernel(ctr_preds, ctr_feats, gt_boxes, points, gt_labels, box_w1, box_b1, box_gamma, box_beta, box_w2, box_b2, cls_w1, cls_b1, cls_gamma, cls_beta, cls_w2, cls_b2, mean_size):
    raise NotImplementedError("write your pallas kernel here")


# speedup vs baseline: 1.5849x; 1.5849x over previous
import numpy as np
import jax
import jax.numpy as jnp
from jax.experimental import pallas as pl
from jax.experimental.pallas import tpu as pltpu

_BIN_SIZE = 12
_BIN_INTER = 2.0 * np.pi / _BIN_SIZE
_BN_INV = 1.0 / np.sqrt(1.0 + 1e-5)


def _head_decode_kernel(
    xyz_ref,
    feats_ref,
    bw1_ref, bb1_ref, bg_ref, bbe_ref, bw2_ref, bb2_ref,
    cw1_ref, cb1_ref, cg_ref, cbe_ref, cw2_ref, cb2_ref,
    ms_ref,
    out_ref,
):
    feats = feats_ref[0]

    hb = jnp.dot(bw1_ref[:], feats, preferred_element_type=jnp.float32)
    hb = (hb + bb1_ref[:]) * (bg_ref[:] * _BN_INV) + bbe_ref[:]
    hb = jnp.maximum(hb, 0.0)
    box_enc = jnp.dot(bw2_ref[:], hb, preferred_element_type=jnp.float32)
    box_enc = box_enc + bb2_ref[:]

    hc = jnp.dot(cw1_ref[:], feats, preferred_element_type=jnp.float32)
    hc = (hc + cb1_ref[:]) * (cg_ref[:] * _BN_INV) + cbe_ref[:]
    hc = jnp.maximum(hc, 0.0)
    clsv = jnp.dot(cw2_ref[:], hc, preferred_element_type=jnp.float32)
    clsv = clsv + cb2_ref[:]

    c0, c1, c2 = clsv[0:1], clsv[1:2], clsv[2:3]
    cls_idx = jnp.where(c1 > c0, 1, 0)
    cls_idx = jnp.where(c2 > jnp.maximum(c0, c1), 2, cls_idx)

    def _anchor(col):
        return jnp.where(
            cls_idx == 0, ms_ref[0, col],
            jnp.where(cls_idx == 1, ms_ref[1, col], ms_ref[2, col]))

    dxa, dya, dza = _anchor(0), _anchor(1), _anchor(2)
    diagonal = jnp.sqrt(dxa * dxa + dya * dya)

    xyz = xyz_ref[0]
    xg = box_enc[0:1] * diagonal + xyz[0:1]
    yg = box_enc[1:2] * diagonal + xyz[1:2]
    zg = box_enc[2:3] * dza + xyz[2:3]
    dxg = jnp.exp(box_enc[3:4]) * dxa
    dyg = jnp.exp(box_enc[4:5]) * dya
    dzg = jnp.exp(box_enc[5:6]) * dza

    best = box_enc[6:7]
    bid = jnp.zeros_like(best, dtype=jnp.int32)
    res = box_enc[18:19]
    for i in range(1, _BIN_SIZE):
        cur = box_enc[6 + i:7 + i]
        gt = cur > best
        bid = jnp.where(gt, i, bid)
        res = jnp.where(gt, box_enc[18 + i:19 + i], res)
        best = jnp.maximum(best, cur)
    rg = (bid.astype(jnp.float32) * _BIN_INTER - np.pi + _BIN_INTER / 2.0
          + res * (_BIN_INTER / 2.0))

    out_ref[0] = jnp.concatenate([xg, yg, zg, dxg, dyg, dzg, rg], axis=0)


def kernel(ctr_preds, ctr_feats, gt_boxes, points, gt_labels, box_w1, box_b1,
           box_gamma, box_beta, box_w2, box_b2, cls_w1, cls_b1, cls_gamma,
           cls_beta, cls_w2, cls_b2, mean_size):
    B, C, K = ctr_feats.shape
    xyz = jnp.transpose(ctr_preds, (0, 2, 1))

    col = lambda v: v[:, None]
    full = pl.BlockSpec(index_map=lambda b: (0, 0))
    batch3 = lambda d: pl.BlockSpec((1, d, K), lambda b: (b, 0, 0))

    out = pl.pallas_call(
        _head_decode_kernel,
        grid=(B,),
        in_specs=[
            batch3(3),
            batch3(C),
            full, full, full, full, full, full,
            full, full, full, full, full, full,
            pl.BlockSpec(memory_space=pltpu.SMEM),
        ],
        out_specs=batch3(7),
        out_shape=jax.ShapeDtypeStruct((B, 7, K), jnp.float32),
    )(xyz, ctr_feats,
      box_w1, col(box_b1), col(box_gamma), col(box_beta), box_w2, col(box_b2),
      cls_w1, col(cls_b1), col(cls_gamma), col(cls_beta), cls_w2, col(cls_b2),
      mean_size)

    return jnp.transpose(out, (0, 2, 1))


# speedup vs baseline: 2.5018x; 1.5785x over previous
import numpy as np
import jax
import jax.numpy as jnp
from jax.experimental import pallas as pl
from jax.experimental.pallas import tpu as pltpu

_BIN_SIZE = 12
_BIN_INTER = 2.0 * np.pi / _BIN_SIZE
_BN_INV = 1.0 / np.sqrt(1.0 + 1e-5)


def _colv(v):
    return jnp.transpose(v.reshape(1, -1))


def _head(feats, w1_ref, b1_ref, g_ref, be_ref, w2_ref, b2_ref):
    h = jnp.dot(w1_ref[:], feats, preferred_element_type=jnp.float32)
    h = (h + _colv(b1_ref[:])) * (_colv(g_ref[:]) * _BN_INV) + _colv(be_ref[:])
    h = jnp.maximum(h, 0.0)
    out = jnp.dot(w2_ref[:], h, preferred_element_type=jnp.float32)
    return out + _colv(b2_ref[:])


def _head_decode_kernel(
    pts_ref,
    feats_ref,
    bw1_ref, bb1_ref, bg_ref, bbe_ref, bw2_ref, bb2_ref,
    cw1_ref, cb1_ref, cg_ref, cbe_ref, cw2_ref, cb2_ref,
    ms_ref,
    out_ref,
):
    feats = feats_ref[0]
    box_enc = _head(feats, bw1_ref, bb1_ref, bg_ref, bbe_ref,
                    bw2_ref, bb2_ref)
    clsv = _head(feats, cw1_ref, cb1_ref, cg_ref, cbe_ref,
                 cw2_ref, cb2_ref)

    c0, c1, c2 = clsv[0:1], clsv[1:2], clsv[2:3]
    cls_idx = jnp.where(c1 > c0, 1, 0)
    cls_idx = jnp.where(c2 > jnp.maximum(c0, c1), 2, cls_idx)

    def _anchor(col):
        return jnp.where(
            cls_idx == 0, ms_ref[0, col],
            jnp.where(cls_idx == 1, ms_ref[1, col], ms_ref[2, col]))

    dxa, dya, dza = _anchor(0), _anchor(1), _anchor(2)
    diagonal = jnp.sqrt(dxa * dxa + dya * dya)

    xyz = jnp.transpose(pts_ref[0])
    xg = box_enc[0:1] * diagonal + xyz[0:1]
    yg = box_enc[1:2] * diagonal + xyz[1:2]
    zg = box_enc[2:3] * dza + xyz[2:3]
    dxg = jnp.exp(box_enc[3:4]) * dxa
    dyg = jnp.exp(box_enc[4:5]) * dya
    dzg = jnp.exp(box_enc[5:6]) * dza

    best = box_enc[6:7]
    bid = jnp.zeros_like(best, dtype=jnp.int32)
    res = box_enc[18:19]
    for i in range(1, _BIN_SIZE):
        cur = box_enc[6 + i:7 + i]
        gt = cur > best
        bid = jnp.where(gt, i, bid)
        res = jnp.where(gt, box_enc[18 + i:19 + i], res)
        best = jnp.maximum(best, cur)
    rg = (bid.astype(jnp.float32) * _BIN_INTER - np.pi + _BIN_INTER / 2.0
          + res * (_BIN_INTER / 2.0))

    rows = jnp.concatenate([xg, yg, zg, dxg, dyg, dzg, rg], axis=0)
    out_ref[0] = jnp.transpose(rows)


def kernel(ctr_preds, ctr_feats, gt_boxes, points, gt_labels, box_w1, box_b1,
           box_gamma, box_beta, box_w2, box_b2, cls_w1, cls_b1, cls_gamma,
           cls_beta, cls_w2, cls_b2, mean_size):
    B, C, K = ctr_feats.shape

    full = pl.BlockSpec(index_map=lambda b: tuple([0]))
    full2 = pl.BlockSpec(index_map=lambda b: (0, 0))
    batch3 = lambda d2, d3: pl.BlockSpec((1, d2, d3), lambda b: (b, 0, 0))

    return pl.pallas_call(
        _head_decode_kernel,
        grid=(B,),
        in_specs=[
            batch3(K, 3),
            batch3(C, K),
            full2, full, full, full, full2, full,
            full2, full, full, full, full2, full,
            pl.BlockSpec(memory_space=pltpu.SMEM),
        ],
        out_specs=batch3(K, 7),
        out_shape=jax.ShapeDtypeStruct((B, K, 7), jnp.float32),
    )(ctr_preds, ctr_feats,
      box_w1, box_b1, box_gamma, box_beta, box_w2, box_b2,
      cls_w1, cls_b1, cls_gamma, cls_beta, cls_w2, cls_b2,
      mean_size)


# speedup vs baseline: 3.6090x; 1.4426x over previous
import numpy as np
import jax
import jax.numpy as jnp
from jax.experimental import pallas as pl
from jax.experimental.pallas import tpu as pltpu

_BIN_SIZE = 12
_BIN_INTER = 2.0 * np.pi / _BIN_SIZE
_BN_INV = 1.0 / np.sqrt(1.0 + 1e-5)


def _colv(v):
    return jnp.transpose(v.reshape(1, -1))


def _head_decode_kernel(
    pts_ref,
    feats_ref,
    bw1_ref, bb1_ref, bg_ref, bbe_ref, bw2_ref, bb2_ref,
    cw1_ref, cb1_ref, cg_ref, cbe_ref, cw2_ref, cb2_ref,
    ms_ref,
    out_ref,
):
    B, C, K = feats_ref.shape
    N = B * K
    feats = jnp.concatenate([feats_ref[b] for b in range(B)], axis=1)

    w1 = jnp.concatenate([bw1_ref[:], cw1_ref[:]], axis=0)
    b1c = _colv(jnp.concatenate([bb1_ref[:], cb1_ref[:]]))
    gc = _colv(jnp.concatenate([bg_ref[:], cg_ref[:]]))
    bec = _colv(jnp.concatenate([bbe_ref[:], cbe_ref[:]]))
    h = jnp.dot(w1, feats, preferred_element_type=jnp.float32)
    h = jnp.maximum((h + b1c) * (gc * _BN_INV) + bec, 0.0)

    M = h.shape[0] // 2
    box_enc = jnp.dot(bw2_ref[:], h[:M],
                      preferred_element_type=jnp.float32) + _colv(bb2_ref[:])
    clsv = jnp.dot(cw2_ref[:], h[M:],
                   preferred_element_type=jnp.float32) + _colv(cb2_ref[:])

    c0, c1, c2 = clsv[0:1], clsv[1:2], clsv[2:3]
    cls_idx = jnp.where(c1 > c0, 1, 0)
    cls_idx = jnp.where(c2 > jnp.maximum(c0, c1), 2, cls_idx)

    def _anchor(col):
        return jnp.where(
            cls_idx == 0, ms_ref[0, col],
            jnp.where(cls_idx == 1, ms_ref[1, col], ms_ref[2, col]))

    dxa, dya, dza = _anchor(0), _anchor(1), _anchor(2)
    diagonal = jnp.sqrt(dxa * dxa + dya * dya)

    xyz = jnp.transpose(pts_ref[:].reshape(N, 3))
    xg = box_enc[0:1] * diagonal + xyz[0:1]
    yg = box_enc[1:2] * diagonal + xyz[1:2]
    zg = box_enc[2:3] * dza + xyz[2:3]
    dxg = jnp.exp(box_enc[3:4]) * dxa
    dyg = jnp.exp(box_enc[4:5]) * dya
    dzg = jnp.exp(box_enc[5:6]) * dza

    best = box_enc[6:7]
    bid = jnp.zeros_like(best, dtype=jnp.int32)
    res = box_enc[18:19]
    for i in range(1, _BIN_SIZE):
        cur = box_enc[6 + i:7 + i]
        gt = cur > best
        bid = jnp.where(gt, i, bid)
        res = jnp.where(gt, box_enc[18 + i:19 + i], res)
        best = jnp.maximum(best, cur)
    rg = (bid.astype(jnp.float32) * _BIN_INTER - np.pi + _BIN_INTER / 2.0
          + res * (_BIN_INTER / 2.0))

    rows = jnp.concatenate([xg, yg, zg, dxg, dyg, dzg, rg], axis=0)
    out_ref[:] = jnp.transpose(rows).reshape(B, K, 7)


def kernel(ctr_preds, ctr_feats, gt_boxes, points, gt_labels, box_w1, box_b1,
           box_gamma, box_beta, box_w2, box_b2, cls_w1, cls_b1, cls_gamma,
           cls_beta, cls_w2, cls_b2, mean_size):
    B, C, K = ctr_feats.shape
    vmem = pl.BlockSpec(memory_space=pltpu.VMEM)
    return pl.pallas_call(
        _head_decode_kernel,
        in_specs=[vmem] * 14 + [pl.BlockSpec(memory_space=pltpu.SMEM)],
        out_specs=vmem,
        out_shape=jax.ShapeDtypeStruct((B, K, 7), jnp.float32),
    )(ctr_preds, ctr_feats,
      box_w1, box_b1, box_gamma, box_beta, box_w2, box_b2,
      cls_w1, cls_b1, cls_gamma, cls_beta, cls_w2, cls_b2,
      mean_size)
